# baseline (speedup 1.0000x reference)
import functools

import jax
import jax.numpy as jnp
from jax import lax
from jax.experimental import pallas as pl
from jax.experimental.pallas import tpu as pltpu
from jax.experimental.pallas import tpu_sc as plsc

NUM_CLASSES = 1000
N_ROWS = 1024
N_COLS = 26
NC = 2
NS = 16
NW = NC * NS
SLABS_PER_W = N_ROWS // NW

_mesh = plsc.VectorSubcoreMesh(core_axis_name="c", subcore_axis_name="s")


def _sc_body(x_hbm, out_hbm, idx_v, buf, sems):
    wid = lax.axis_index("s") * NC + lax.axis_index("c")
    base = wid * SLABS_PER_W
    pltpu.sync_copy(x_hbm.at[pl.ds(base, SLABS_PER_W)], idx_v)

    lanes = lax.iota(jnp.int32, 16)
    zeros = jnp.zeros((16,), jnp.float32)

    def _zero_row(r, carry):
        def _zero_chunk(c, carry2):
            buf[r, pl.ds(c * 16, 16)] = zeros
            return carry2

        lax.fori_loop(0, 62, _zero_chunk, 0)
        plsc.store_scatter(
            buf,
            [jnp.full((16,), r, jnp.int32), 992 + lanes],
            zeros,
            mask=lanes < 8,
        )
        return carry

    lax.fori_loop(0, 2 * N_COLS, _zero_row, 0)

    def _scatter(slab, b, val):
        lo = idx_v[slab, pl.ds(0, 16)]
        hi = idx_v[slab, pl.ds(10, 16)]
        vv = jnp.full((16,), val, jnp.float32)
        plsc.store_scatter(buf, [b * N_COLS + lanes, lo], vv)
        plsc.store_scatter(buf, [b * N_COLS + 10 + lanes, hi], vv)

    def _dma(i, b):
        return pltpu.make_async_copy(
            buf.at[pl.ds(b * N_COLS, N_COLS)], out_hbm.at[base + i], sems.at[b]
        )

    for i in range(SLABS_PER_W):
        b = i % 2
        if i >= 2:
            _dma(i - 2, b).wait()
            _scatter(i - 2, b, 0.0)
        _scatter(i, b, 1.0)
        _dma(i, b).start()

    for i in (SLABS_PER_W - 2, SLABS_PER_W - 1):
        _dma(i, i % 2).wait()


_sc_onehot = pl.kernel(
    _sc_body,
    out_type=jax.ShapeDtypeStruct((N_ROWS, N_COLS, NUM_CLASSES), jnp.float32),
    mesh=_mesh,
    scratch_types=[
        pltpu.VMEM((SLABS_PER_W, N_COLS), jnp.int32),
        pltpu.VMEM((2 * N_COLS, NUM_CLASSES), jnp.float32),
        pltpu.SemaphoreType.DMA((2,)),
    ],
    compiler_params=pltpu.CompilerParams(
        use_tc_tiling_on_sc=False, needs_layout_passes=False
    ),
)


def kernel(x):
    return _sc_onehot(x)


# speedup vs baseline: 1.8141x; 1.8141x over previous
import functools

import jax
import jax.numpy as jnp
from jax import lax
from jax.experimental import pallas as pl
from jax.experimental.pallas import tpu as pltpu
from jax.experimental.pallas import tpu_sc as plsc

NUM_CLASSES = 1000
N_ROWS = 1024
N_COLS = 26
NC = 2
NS = 16
NW = NC * NS
SLABS_PER_W = N_ROWS // NW

_mesh = plsc.VectorSubcoreMesh(core_axis_name="c", subcore_axis_name="s")


def _sc_body(x_hbm, out_hbm, idx_v, buf0, buf1, sems):
    wid = lax.axis_index("s") * NC + lax.axis_index("c")
    base = wid * SLABS_PER_W
    pltpu.sync_copy(x_hbm.at[pl.ds(base, SLABS_PER_W)], idx_v)

    bufs = (buf0, buf1)
    lanes = lax.iota(jnp.int32, 16)
    zeros = jnp.zeros((16,), jnp.float32)

    def _zero_row(r, carry):
        def _zero_chunk(c, carry2):
            buf0[r, pl.ds(c * 16, 16)] = zeros
            buf1[r, pl.ds(c * 16, 16)] = zeros
            return carry2

        lax.fori_loop(0, 62, _zero_chunk, 0)
        rvec = jnp.full((16,), r, jnp.int32)
        tail = lanes < 8
        plsc.store_scatter(buf0, [rvec, 992 + lanes], zeros, mask=tail)
        plsc.store_scatter(buf1, [rvec, 992 + lanes], zeros, mask=tail)
        return carry

    lax.fori_loop(0, N_COLS, _zero_row, 0)

    def _scatter(slab, b, val):
        lo = idx_v[slab, pl.ds(0, 16)]
        hi = idx_v[slab, pl.ds(10, 16)]
        vv = jnp.full((16,), val, jnp.float32)
        plsc.store_scatter(bufs[b], [lanes, lo], vv)
        plsc.store_scatter(bufs[b], [10 + lanes, hi], vv)

    def _dma(i, b):
        return pltpu.make_async_copy(bufs[b], out_hbm.at[base + i], sems.at[b])

    for i in range(SLABS_PER_W):
        b = i % 2
        if i >= 2:
            _dma(i - 2, b).wait()
            _scatter(i - 2, b, 0.0)
        _scatter(i, b, 1.0)
        _dma(i, b).start()

    for i in (SLABS_PER_W - 2, SLABS_PER_W - 1):
        _dma(i, i % 2).wait()


_sc_onehot = pl.kernel(
    _sc_body,
    out_type=jax.ShapeDtypeStruct((N_ROWS, N_COLS, NUM_CLASSES), jnp.float32),
    mesh=_mesh,
    scratch_types=[
        pltpu.VMEM((SLABS_PER_W, N_COLS), jnp.int32),
        pltpu.VMEM((N_COLS, NUM_CLASSES), jnp.float32),
        pltpu.VMEM((N_COLS, NUM_CLASSES), jnp.float32),
        pltpu.SemaphoreType.DMA((2,)),
    ],
    compiler_params=pltpu.CompilerParams(
        use_tc_tiling_on_sc=True, needs_layout_passes=False
    ),
)


def kernel(x):
    return _sc_onehot(x)


# speedup vs baseline: 1.8631x; 1.0270x over previous
import jax
import jax.numpy as jnp
from jax import lax
from jax.experimental import pallas as pl
from jax.experimental.pallas import tpu as pltpu

NUM_CLASSES = 1000
ROWS_PER_BLOCK = 32
NBUF = 6


def _onehot_manual(x_ref, o_hbm, buf, sems):
    i = pl.program_id(0)
    slot = lax.rem(i, NBUF)

    @pl.when(i >= NBUF)
    def _():
        pltpu.make_async_copy(
            buf.at[slot],
            o_hbm.at[pl.ds((i - NBUF) * ROWS_PER_BLOCK, ROWS_PER_BLOCK)],
            sems.at[slot],
        ).wait()

    idx = x_ref[...]
    iota = lax.broadcasted_iota(
        jnp.int32, (ROWS_PER_BLOCK, x_ref.shape[1], NUM_CLASSES), 2
    )
    buf[slot] = (idx == iota).astype(jnp.float32)
    for p in range(NBUF):
        @pl.when(slot == p)
        def _():
            pltpu.async_copy(
                buf.at[p],
                o_hbm.at[pl.ds(i * ROWS_PER_BLOCK, ROWS_PER_BLOCK)],
                sems.at[p],
                priority=p % 2,
            )

    ng = pl.num_programs(0)

    @pl.when(i == ng - 1)
    def _():
        for k in range(NBUF):
            step = ng - NBUF + k
            s = step % NBUF
            pltpu.make_async_copy(
                buf.at[s],
                o_hbm.at[pl.ds(step * ROWS_PER_BLOCK, ROWS_PER_BLOCK)],
                sems.at[s],
            ).wait()


def kernel(x):
    n, m = x.shape
    grid = n // ROWS_PER_BLOCK
    return pl.pallas_call(
        _onehot_manual,
        grid=(grid,),
        in_specs=[pl.BlockSpec((ROWS_PER_BLOCK, m, 1), lambda i: (i, 0, 0))],
        out_specs=pl.BlockSpec(memory_space=pl.ANY),
        out_shape=jax.ShapeDtypeStruct((n, m, NUM_CLASSES), jnp.float32),
        scratch_shapes=[
            pltpu.VMEM((NBUF, ROWS_PER_BLOCK, m, NUM_CLASSES), jnp.float32),
            pltpu.SemaphoreType.DMA((NBUF,)),
        ],
    )(x[:, :, None])
